# Initial kernel scaffold; baseline (speedup 1.0000x reference)
import jax
import jax.numpy as jnp
from jax.experimental import pallas as pl


def kernel(x, d, sigma_uvw, beta):
    raise NotImplementedError("write your pallas kernel here")


# speedup vs baseline: 3.5889x; 3.5889x over previous
import functools

import jax
import jax.numpy as jnp
from jax import lax
from jax.experimental import pallas as pl
from jax.experimental.pallas import tpu as pltpu
from jax.experimental.pallas import tpu_sc as plsc

SCALE = 4.0
NP = 128
ND = 128
D = 8
ROW = 1 + 3 * D

NC = 2
NS = 16
NW = NC * NS
L = 16

C = 128


NGRAN = NP * NP * NP * ROW // 16


def _sc_body(xf, df, gview, bsm, colorf, s0f,
             xbuf, dbuf, ibuf, ubuf, mbuf, obuf, subuf, bbuf, cbuf, sbuf,
             semg, semb):
    n = s0f.shape[0]
    pts_per_tile = n // NW
    n_chunks = pts_per_tile // C
    wid = lax.axis_index("s") * NC + lax.axis_index("c")
    base0 = wid * pts_per_tile
    lanes = lax.iota(jnp.int32, L)

    def chunk_body(g, carry):
        base = base0 + g * C
        pltpu.sync_copy(xf.at[pl.ds(base * 3, C * 3)], xbuf)
        pltpu.sync_copy(df.at[pl.ds(base * 3, C * 3)], dbuf)
        for sb in range(C // L):
            rows3 = (sb * L + lanes) * 3
            x0 = plsc.load_gather(xbuf, [rows3])
            x1 = plsc.load_gather(xbuf, [rows3 + 1])
            x2 = plsc.load_gather(xbuf, [rows3 + 2])
            d0 = plsc.load_gather(dbuf, [rows3])
            d1 = plsc.load_gather(dbuf, [rows3 + 1])
            inside = ((jnp.abs(x0) < SCALE / 2) & (jnp.abs(x1) < SCALE / 2)
                      & (jnp.abs(x2) < SCALE / 2))
            maskf = jnp.where(inside, jnp.float32(1.0), jnp.float32(0.0))
            iv0 = jnp.clip((x0 * (NP / SCALE) + NP / 2).astype(jnp.int32), 0, NP - 1)
            iv1 = jnp.clip((x1 * (NP / SCALE) + NP / 2).astype(jnp.int32), 0, NP - 1)
            iv2 = jnp.clip((x2 * (NP / SCALE) + NP / 2).astype(jnp.int32), 0, NP - 1)
            v = (iv0 * NP + iv1) * NP + iv2
            w = v * ROW
            gr = lax.shift_right_logical(w, 4)
            obuf[pl.ds(sb * L, L)] = w & 15
            for k in range(3):
                gk = jnp.minimum(gr + k, jnp.int32(NGRAN - 1))
                plsc.store_scatter(ibuf, [rows3 + k], gk)
            u0 = jnp.clip((d0 * ND).astype(jnp.int32), 0, ND - 1)
            u1 = jnp.clip((d1 * ND).astype(jnp.int32), 0, ND - 1)
            ubuf[pl.ds(sb * L, L)] = u0 * ND + u1
            mbuf[pl.ds(sb * L, L)] = maskf
        cp1 = pltpu.async_copy(gview.at[ibuf], subuf, semg)
        cp2 = pltpu.async_copy(bsm.at[ubuf], bbuf, semb)
        cp1.wait()
        cp2.wait()
        for sb in range(C // L):
            rows = sb * L + lanes
            m = mbuf[pl.ds(sb * L, L)]
            fp = (rows * 3) * L + obuf[pl.ds(sb * L, L)]
            def su_col(j):
                pos = fp + j
                return plsc.load_gather(
                    subuf, [lax.shift_right_logical(pos, 4), pos & 15])
            s0 = su_col(0)
            sbuf[pl.ds(sb * L, L)] = jnp.where(m > 0.5, s0, jnp.float32(-1e30))
            bj = [plsc.load_gather(bbuf, [rows, jnp.full((L,), j, jnp.int32)])
                  for j in range(D)]
            for c in range(3):
                acc = jnp.zeros((L,), jnp.float32)
                for j in range(D):
                    sg = 1.0 / (1.0 + jnp.exp(-su_col(1 + c * D + j)))
                    acc = acc + sg * bj[j]
                plsc.store_scatter(cbuf, [rows * 3 + c], acc * m)
        pltpu.sync_copy(cbuf, colorf.at[pl.ds(base * 3, C * 3)])
        pltpu.sync_copy(sbuf, s0f.at[pl.ds(base, C)])
        return carry

    lax.fori_loop(0, n_chunks, chunk_body, 0)


@functools.partial(jax.jit, static_argnums=())
def _sc_call(xf, df, table, bsm):
    n = xf.shape[0] // 3
    mesh = plsc.VectorSubcoreMesh(core_axis_name="c", subcore_axis_name="s")
    return pl.kernel(
        _sc_body,
        out_type=[
            jax.ShapeDtypeStruct((n * 3,), jnp.float32),
            jax.ShapeDtypeStruct((n,), jnp.float32),
        ],
        mesh=mesh,
        compiler_params=pltpu.CompilerParams(
            needs_layout_passes=False, use_tc_tiling_on_sc=False),
        scratch_types=[
            pltpu.VMEM((C * 3,), jnp.float32),
            pltpu.VMEM((C * 3,), jnp.float32),
            pltpu.VMEM((C * 3,), jnp.int32),
            pltpu.VMEM((C,), jnp.int32),
            pltpu.VMEM((C,), jnp.float32),
            pltpu.VMEM((C,), jnp.int32),
            pltpu.VMEM((C * 3, 16), jnp.float32),
            pltpu.VMEM((C, D), jnp.float32),
            pltpu.VMEM((C * 3,), jnp.float32),
            pltpu.VMEM((C,), jnp.float32),
            pltpu.SemaphoreType.DMA,
            pltpu.SemaphoreType.DMA,
        ],
    )(xf, df, table, bsm)


def _softmax_tc(beta2):
    rows = beta2.shape[0]
    blk = 2048

    def body(b_ref, o_ref):
        b = b_ref[...]
        mx = jnp.max(b, axis=-1, keepdims=True)
        e = jnp.exp(b - mx)
        o_ref[...] = e / jnp.sum(e, axis=-1, keepdims=True)

    return pl.pallas_call(
        body,
        grid=(rows // blk,),
        in_specs=[pl.BlockSpec((blk, D), lambda i: (i, 0))],
        out_specs=pl.BlockSpec((blk, D), lambda i: (i, 0)),
        out_shape=jax.ShapeDtypeStruct((rows, D), jnp.float32),
    )(beta2)


def _softplus_tc(s0):
    n = s0.shape[0]
    cols = 1024
    rows = n // cols
    blk = rows // 8

    def body(z_ref, o_ref):
        z = z_ref[...]
        o_ref[...] = jnp.maximum(z, 0.0) + jnp.log1p(jnp.exp(-jnp.abs(z)))

    out = pl.pallas_call(
        body,
        grid=(8,),
        in_specs=[pl.BlockSpec((blk, cols), lambda i: (i, 0))],
        out_specs=pl.BlockSpec((blk, cols), lambda i: (i, 0)),
        out_shape=jax.ShapeDtypeStruct((rows, cols), jnp.float32),
    )(s0.reshape(rows, cols))
    return out.reshape(n, 1)


def kernel(x, d, sigma_uvw, beta):
    n = x.shape[0]
    gview = sigma_uvw.reshape(NGRAN, 16)
    beta2 = beta.reshape(ND * ND, D)
    bsm = _softmax_tc(beta2)
    colorf, s0 = _sc_call(x.reshape(-1), d.reshape(-1), gview, bsm)
    color = colorf.reshape(n, 3)
    sigma = _softplus_tc(s0)
    return (color, sigma)


# speedup vs baseline: 7.0497x; 1.9643x over previous
import functools

import jax
import jax.numpy as jnp
from jax import lax
from jax.experimental import pallas as pl
from jax.experimental.pallas import tpu as pltpu
from jax.experimental.pallas import tpu_sc as plsc

SCALE = 4.0
NP = 128
ND = 128
D = 8
ROW = 1 + 3 * D

NC = 2
NS = 16
NW = NC * NS
L = 16

C = 128
NGRAN = NP * NP * NP * ROW // 16


def _sc_body(x0f, x1f, x2f, d0f, d1f, gview, bsm,
             c0f, c1f, c2f, s0f,
             xb0, xb1, xb2, db0, db1, ibuf, ubuf, mbuf, obuf,
             subuf, bbuf, cb0, cb1, cb2, sb_out,
             semg, semb):
    n = s0f.shape[0]
    pts_per_tile = n // NW
    n_chunks = pts_per_tile // C
    wid = lax.axis_index("s") * NC + lax.axis_index("c")
    base0 = wid * pts_per_tile
    lanes = lax.iota(jnp.int32, L)

    def chunk_body(g, carry):
        base = base0 + g * C
        pltpu.sync_copy(x0f.at[pl.ds(base, C)], xb0)
        pltpu.sync_copy(x1f.at[pl.ds(base, C)], xb1)
        pltpu.sync_copy(x2f.at[pl.ds(base, C)], xb2)
        pltpu.sync_copy(d0f.at[pl.ds(base, C)], db0)
        pltpu.sync_copy(d1f.at[pl.ds(base, C)], db1)
        for sb in range(C // L):
            sl = pl.ds(sb * L, L)
            x0 = xb0[sl]
            x1 = xb1[sl]
            x2 = xb2[sl]
            d0 = db0[sl]
            d1 = db1[sl]
            inside = ((jnp.abs(x0) < SCALE / 2) & (jnp.abs(x1) < SCALE / 2)
                      & (jnp.abs(x2) < SCALE / 2))
            maskf = jnp.where(inside, jnp.float32(1.0), jnp.float32(0.0))
            iv0 = jnp.clip((x0 * (NP / SCALE) + NP / 2).astype(jnp.int32), 0, NP - 1)
            iv1 = jnp.clip((x1 * (NP / SCALE) + NP / 2).astype(jnp.int32), 0, NP - 1)
            iv2 = jnp.clip((x2 * (NP / SCALE) + NP / 2).astype(jnp.int32), 0, NP - 1)
            v = (iv0 * NP + iv1) * NP + iv2
            w = v * ROW
            gr = lax.shift_right_logical(w, 4)
            obuf[sl] = w & 15
            rows3 = (sb * L + lanes) * 3
            for k in range(3):
                gk = jnp.minimum(gr + k, jnp.int32(NGRAN - 1))
                plsc.store_scatter(ibuf, [rows3 + k], gk)
            u0 = jnp.clip((d0 * ND).astype(jnp.int32), 0, ND - 1)
            u1 = jnp.clip((d1 * ND).astype(jnp.int32), 0, ND - 1)
            ubuf[sl] = u0 * ND + u1
            mbuf[sl] = maskf
        cp1 = pltpu.async_copy(gview.at[ibuf], subuf, semg)
        cp2 = pltpu.async_copy(bsm.at[ubuf], bbuf, semb)
        cp1.wait()
        cp2.wait()
        for sb in range(C // L):
            sl = pl.ds(sb * L, L)
            rows = sb * L + lanes
            m = mbuf[sl]
            fp = rows * (3 * L) + obuf[sl]
            def su_col(j):
                pos = fp + j
                return plsc.load_gather(
                    subuf, [lax.shift_right_logical(pos, 4), pos & 15])
            s0 = su_col(0)
            sb_out[sl] = jnp.where(m > 0.5, s0, jnp.float32(-1e30))
            bj = [plsc.load_gather(bbuf, [rows, jnp.full((L,), j, jnp.int32)])
                  for j in range(D)]
            for c, cb in ((0, cb0), (1, cb1), (2, cb2)):
                acc = jnp.zeros((L,), jnp.float32)
                for j in range(D):
                    sg = 1.0 / (1.0 + jnp.exp(-su_col(1 + c * D + j)))
                    acc = acc + sg * bj[j]
                cb[sl] = acc * m
        pltpu.sync_copy(cb0, c0f.at[pl.ds(base, C)])
        pltpu.sync_copy(cb1, c1f.at[pl.ds(base, C)])
        pltpu.sync_copy(cb2, c2f.at[pl.ds(base, C)])
        pltpu.sync_copy(sb_out, s0f.at[pl.ds(base, C)])
        return carry

    lax.fori_loop(0, n_chunks, chunk_body, 0)


def _sc_call(x0, x1, x2, d0, d1, gview, bsm):
    n = x0.shape[0]
    mesh = plsc.VectorSubcoreMesh(core_axis_name="c", subcore_axis_name="s")
    f32 = jnp.float32
    return pl.kernel(
        _sc_body,
        out_type=[jax.ShapeDtypeStruct((n,), f32) for _ in range(4)],
        mesh=mesh,
        compiler_params=pltpu.CompilerParams(
            needs_layout_passes=False, use_tc_tiling_on_sc=False),
        scratch_types=[
            pltpu.VMEM((C,), f32),
            pltpu.VMEM((C,), f32),
            pltpu.VMEM((C,), f32),
            pltpu.VMEM((C,), f32),
            pltpu.VMEM((C,), f32),
            pltpu.VMEM((C * 3,), jnp.int32),
            pltpu.VMEM((C,), jnp.int32),
            pltpu.VMEM((C,), f32),
            pltpu.VMEM((C,), jnp.int32),
            pltpu.VMEM((C * 3, 16), f32),
            pltpu.VMEM((C, D), f32),
            pltpu.VMEM((C,), f32),
            pltpu.VMEM((C,), f32),
            pltpu.VMEM((C,), f32),
            pltpu.VMEM((C,), f32),
            pltpu.SemaphoreType.DMA,
            pltpu.SemaphoreType.DMA,
        ],
    )(x0, x1, x2, d0, d1, gview, bsm)


def _softmax_tc(beta2):
    rows = beta2.shape[0]
    blk = 2048

    def body(b_ref, o_ref):
        b = b_ref[...]
        mx = jnp.max(b, axis=-1, keepdims=True)
        e = jnp.exp(b - mx)
        o_ref[...] = e / jnp.sum(e, axis=-1, keepdims=True)

    return pl.pallas_call(
        body,
        grid=(rows // blk,),
        in_specs=[pl.BlockSpec((blk, D), lambda i: (i, 0))],
        out_specs=pl.BlockSpec((blk, D), lambda i: (i, 0)),
        out_shape=jax.ShapeDtypeStruct((rows, D), jnp.float32),
    )(beta2)


def _softplus_tc(s0):
    n = s0.shape[0]
    cols = 1024
    rows = n // cols
    blk = rows // 8

    def body(z_ref, o_ref):
        z = z_ref[...]
        o_ref[...] = jnp.maximum(z, 0.0) + jnp.log1p(jnp.exp(-jnp.abs(z)))

    out = pl.pallas_call(
        body,
        grid=(8,),
        in_specs=[pl.BlockSpec((blk, cols), lambda i: (i, 0))],
        out_specs=pl.BlockSpec((blk, cols), lambda i: (i, 0)),
        out_shape=jax.ShapeDtypeStruct((rows, cols), jnp.float32),
    )(s0.reshape(rows, cols))
    return out.reshape(n, 1)


def kernel(x, d, sigma_uvw, beta):
    gview = sigma_uvw.reshape(NGRAN, 16)
    beta2 = beta.reshape(ND * ND, D)
    bsm = _softmax_tc(beta2)
    c0, c1, c2, s0 = _sc_call(
        x[:, 0], x[:, 1], x[:, 2], d[:, 0], d[:, 1], gview, bsm)
    color = jnp.stack([c0, c1, c2], axis=-1)
    sigma = _softplus_tc(s0)
    return (color, sigma)


# speedup vs baseline: 12.4756x; 1.7697x over previous
import functools

import jax
import jax.numpy as jnp
from jax import lax
from jax.experimental import pallas as pl
from jax.experimental.pallas import tpu as pltpu
from jax.experimental.pallas import tpu_sc as plsc

SCALE = 4.0
NP = 128
ND = 128
D = 8
ROW = 1 + 3 * D

NC = 2
NS = 16
NW = NC * NS
L = 16

C = 128
NGRAN = NP * NP * NP * ROW // 16


BS = 1024
CPB = BS // C
NSLOT = 4


def _sc_body(x0f, x1f, x2f, d0f, d1f, gview, bsm,
             c0f, c1f, c2f, s0f,
             xb0, xb1, xb2, db0, db1, ibuf, ubuf, mbuf, obuf,
             subuf, bbuf, cb0, cb1, cb2, sb_out,
             *sems):
    semg = sems[:NSLOT]
    semb = sems[NSLOT:]
    n = s0f.shape[0]
    pts_per_tile = n // NW
    n_blocks = pts_per_tile // BS
    wid = lax.axis_index("s") * NC + lax.axis_index("c")
    base0 = wid * pts_per_tile
    lanes = lax.iota(jnp.int32, L)

    def stage1_chunk(c):
        def body(sb, carry):
            sl = pl.ds(c * C + sb * L, L)
            x0 = xb0[sl]
            x1 = xb1[sl]
            x2 = xb2[sl]
            d0 = db0[sl]
            d1 = db1[sl]
            inside = ((jnp.abs(x0) < SCALE / 2) & (jnp.abs(x1) < SCALE / 2)
                      & (jnp.abs(x2) < SCALE / 2))
            maskf = jnp.where(inside, jnp.float32(1.0), jnp.float32(0.0))
            iv0 = jnp.clip((x0 * (NP / SCALE) + NP / 2).astype(jnp.int32), 0, NP - 1)
            iv1 = jnp.clip((x1 * (NP / SCALE) + NP / 2).astype(jnp.int32), 0, NP - 1)
            iv2 = jnp.clip((x2 * (NP / SCALE) + NP / 2).astype(jnp.int32), 0, NP - 1)
            v = (iv0 * NP + iv1) * NP + iv2
            w = v * ROW
            gr = lax.shift_right_logical(w, 4)
            obuf[sl] = w & 15
            rows3 = (c * C + sb * L + lanes) * 3
            for k in range(3):
                gk = jnp.minimum(gr + k, jnp.int32(NGRAN - 1))
                plsc.store_scatter(ibuf, [rows3 + k], gk)
            u0 = jnp.clip((d0 * ND).astype(jnp.int32), 0, ND - 1)
            u1 = jnp.clip((d1 * ND).astype(jnp.int32), 0, ND - 1)
            ubuf[sl] = u0 * ND + u1
            mbuf[sl] = maskf
            return carry

        lax.fori_loop(0, C // L, body, 0)

    def fire(c, slot):
        cp1 = pltpu.async_copy(
            gview.at[ibuf.at[pl.ds(c * C * 3, C * 3)]], subuf.at[slot],
            semg[slot])
        cp2 = pltpu.async_copy(
            bsm.at[ubuf.at[pl.ds(c * C, C)]], bbuf.at[slot], semb[slot])
        return (cp1, cp2)

    def stage3_chunk(c, slot):
        def body(sb, carry):
            sl = pl.ds(c * C + sb * L, L)
            rows = sb * L + lanes
            m = mbuf[sl]
            fp = rows * (3 * L) + obuf[sl]

            def su_col(j):
                pos = fp + j
                return plsc.load_gather(
                    subuf, [jnp.full((L,), slot, jnp.int32),
                            lax.shift_right_logical(pos, 4), pos & 15])

            s0 = su_col(0)
            sb_out[sl] = jnp.where(m > 0.5, s0, jnp.float32(-1e30))
            bj = [plsc.load_gather(
                      bbuf, [jnp.full((L,), slot, jnp.int32), rows,
                             jnp.full((L,), j, jnp.int32)])
                  for j in range(D)]
            for cc, cb in ((0, cb0), (1, cb1), (2, cb2)):
                acc = jnp.zeros((L,), jnp.float32)
                for j in range(D):
                    sg = 1.0 / (1.0 + jnp.exp(-su_col(1 + cc * D + j)))
                    acc = acc + sg * bj[j]
                cb[sl] = acc * m
            return carry

        lax.fori_loop(0, C // L, body, 0)

    def block_body(b, carry):
        base = base0 + b * BS
        pltpu.sync_copy(x0f.at[pl.ds(base, BS)], xb0)
        pltpu.sync_copy(x1f.at[pl.ds(base, BS)], xb1)
        pltpu.sync_copy(x2f.at[pl.ds(base, BS)], xb2)
        pltpu.sync_copy(d0f.at[pl.ds(base, BS)], db0)
        pltpu.sync_copy(d1f.at[pl.ds(base, BS)], db1)
        for c in range(CPB):
            stage1_chunk(c)
        handles = {}
        for c in range(NSLOT):
            handles[c] = fire(c, c % NSLOT)
        for c in range(CPB):
            cp1, cp2 = handles.pop(c)
            cp1.wait()
            cp2.wait()
            stage3_chunk(c, c % NSLOT)
            if c + NSLOT < CPB:
                handles[c + NSLOT] = fire(c + NSLOT, c % NSLOT)
        pltpu.sync_copy(cb0, c0f.at[pl.ds(base, BS)])
        pltpu.sync_copy(cb1, c1f.at[pl.ds(base, BS)])
        pltpu.sync_copy(cb2, c2f.at[pl.ds(base, BS)])
        pltpu.sync_copy(sb_out, s0f.at[pl.ds(base, BS)])
        return carry

    lax.fori_loop(0, n_blocks, block_body, 0)


def _sc_call(x0, x1, x2, d0, d1, gview, bsm):
    n = x0.shape[0]
    mesh = plsc.VectorSubcoreMesh(core_axis_name="c", subcore_axis_name="s")
    f32 = jnp.float32
    return pl.kernel(
        _sc_body,
        out_type=[jax.ShapeDtypeStruct((n,), f32) for _ in range(4)],
        mesh=mesh,
        compiler_params=pltpu.CompilerParams(
            needs_layout_passes=False, use_tc_tiling_on_sc=False),
        scratch_types=(
            [
                pltpu.VMEM((BS,), f32),
                pltpu.VMEM((BS,), f32),
                pltpu.VMEM((BS,), f32),
                pltpu.VMEM((BS,), f32),
                pltpu.VMEM((BS,), f32),
                pltpu.VMEM((BS * 3,), jnp.int32),
                pltpu.VMEM((BS,), jnp.int32),
                pltpu.VMEM((BS,), f32),
                pltpu.VMEM((BS,), jnp.int32),
                pltpu.VMEM((NSLOT, C * 3, 16), f32),
                pltpu.VMEM((NSLOT, C, D), f32),
                pltpu.VMEM((BS,), f32),
                pltpu.VMEM((BS,), f32),
                pltpu.VMEM((BS,), f32),
                pltpu.VMEM((BS,), f32),
            ]
            + [pltpu.SemaphoreType.DMA] * (2 * NSLOT)
        ),
    )(x0, x1, x2, d0, d1, gview, bsm)


def _softmax_tc(beta2):
    rows = beta2.shape[0]
    blk = 2048

    def body(b_ref, o_ref):
        b = b_ref[...]
        mx = jnp.max(b, axis=-1, keepdims=True)
        e = jnp.exp(b - mx)
        o_ref[...] = e / jnp.sum(e, axis=-1, keepdims=True)

    return pl.pallas_call(
        body,
        grid=(rows // blk,),
        in_specs=[pl.BlockSpec((blk, D), lambda i: (i, 0))],
        out_specs=pl.BlockSpec((blk, D), lambda i: (i, 0)),
        out_shape=jax.ShapeDtypeStruct((rows, D), jnp.float32),
    )(beta2)


def _softplus_tc(s0):
    n = s0.shape[0]
    cols = 1024
    rows = n // cols
    blk = rows // 8

    def body(z_ref, o_ref):
        z = z_ref[...]
        o_ref[...] = jnp.maximum(z, 0.0) + jnp.log1p(jnp.exp(-jnp.abs(z)))

    out = pl.pallas_call(
        body,
        grid=(8,),
        in_specs=[pl.BlockSpec((blk, cols), lambda i: (i, 0))],
        out_specs=pl.BlockSpec((blk, cols), lambda i: (i, 0)),
        out_shape=jax.ShapeDtypeStruct((rows, cols), jnp.float32),
    )(s0.reshape(rows, cols))
    return out.reshape(n, 1)


def kernel(x, d, sigma_uvw, beta):
    gview = sigma_uvw.reshape(NGRAN, 16)
    beta2 = beta.reshape(ND * ND, D)
    bsm = _softmax_tc(beta2)
    c0, c1, c2, s0 = _sc_call(
        x[:, 0], x[:, 1], x[:, 2], d[:, 0], d[:, 1], gview, bsm)
    color = jnp.stack([c0, c1, c2], axis=-1)
    sigma = _softplus_tc(s0)
    return (color, sigma)


# speedup vs baseline: 19.9116x; 1.5960x over previous
import functools

import jax
import jax.numpy as jnp
from jax import lax
from jax.experimental import pallas as pl
from jax.experimental.pallas import tpu as pltpu
from jax.experimental.pallas import tpu_sc as plsc

SCALE = 4.0
NP = 128
ND = 128
D = 8
ROW = 1 + 3 * D

NC = 2
NS = 16
NW = NC * NS
L = 16

C = 128
NGRAN = NP * NP * NP * ROW // 16


BS = 1024
CPB = BS // C
NSLOT = 4


RB = 8
RS = RB * NP
RBLK = NP * NP // RS * 4
RROWS = RS * ROW // 16


def _repack_body(st, tact, tbuf, obuf, semi0, semi1, semo0, semo1):
    wid = lax.axis_index("s") * NC + lax.axis_index("c")
    lanes = lax.iota(jnp.int32, L)
    l25 = lanes * ROW
    semi = (semi0, semi1)
    semo = (semo0, semo1)

    def src_slice(blk):
        a = wid * 4 + lax.div(blk, RBLK // 4)
        b0 = lax.rem(blk, RBLK // 4) * RB
        return st.at[a, :, pl.ds(b0, RB), :]

    def dst_slice(blk):
        r0 = (wid * RBLK + blk) * RROWS
        return tact.at[pl.ds(r0, RROWS)]

    def fire_in(blk, s):
        pltpu.async_copy(src_slice(blk), tbuf.at[s], semi[s])

    def wait_in(s):
        pltpu.make_async_copy(src_slice(0), tbuf.at[s], semi[s]).wait()

    def fire_out(blk, s):
        pltpu.async_copy(obuf.at[s], dst_slice(blk), semo[s])

    def drain_out(s):
        pltpu.make_async_copy(obuf.at[s], dst_slice(0), semo[s]).wait()

    def compute(s):
        def grp_body(grp, carry):
            p0 = grp * L
            pb = jnp.full((L,), lax.shift_right_logical(p0, 7), jnp.int32)
            pc = (p0 & 127) + lanes
            t0 = l25 + p0 * ROW
            for e in range(ROW):
                val = plsc.load_gather(
                    tbuf, [jnp.full((L,), s, jnp.int32),
                           jnp.full((L,), e, jnp.int32), pb, pc])
                t = t0 + e
                plsc.store_scatter(
                    obuf, [jnp.full((L,), s, jnp.int32),
                           lax.shift_right_logical(t, 4), t & 15], val)
            return carry

        lax.fori_loop(0, RS // L, grp_body, 0)

    fire_in(0, 0)

    def pair_body(j, carry):
        blk0 = j * 2
        fire_in(blk0 + 1, 1)
        wait_in(0)

        @pl.when(j > 0)
        def _():
            drain_out(0)

        compute(0)
        fire_out(blk0, 0)

        @pl.when(j < RBLK // 2 - 1)
        def _():
            fire_in(blk0 + 2, 0)

        wait_in(1)

        @pl.when(j > 0)
        def _():
            drain_out(1)

        compute(1)
        fire_out(blk0 + 1, 1)
        return carry

    lax.fori_loop(0, RBLK // 2, pair_body, 0)
    drain_out(0)
    drain_out(1)


def _repack_call(st):
    mesh = plsc.VectorSubcoreMesh(core_axis_name="c", subcore_axis_name="s")
    f32 = jnp.float32
    return pl.kernel(
        _repack_body,
        out_type=[jax.ShapeDtypeStruct((NGRAN, 16), f32)],
        mesh=mesh,
        compiler_params=pltpu.CompilerParams(
            needs_layout_passes=False, use_tc_tiling_on_sc=False),
        scratch_types=[
            pltpu.VMEM((2, ROW, RB, NP), f32),
            pltpu.VMEM((2, RROWS, 16), f32),
            pltpu.SemaphoreType.DMA,
            pltpu.SemaphoreType.DMA,
            pltpu.SemaphoreType.DMA,
            pltpu.SemaphoreType.DMA,
        ],
    )(st)[0]


def _sc_body(x0f, x1f, x2f, d0f, d1f, gview, bsm,
             c0f, c1f, c2f, s0f,
             xb0, xb1, xb2, db0, db1, ibuf, ubuf, mbuf, obuf,
             subuf, bbuf, cb0, cb1, cb2, sb_out,
             *sems):
    semg = sems[:NSLOT]
    semb = sems[NSLOT:]
    n = s0f.shape[0]
    pts_per_tile = n // NW
    n_blocks = pts_per_tile // BS
    wid = lax.axis_index("s") * NC + lax.axis_index("c")
    base0 = wid * pts_per_tile
    lanes = lax.iota(jnp.int32, L)

    def stage1_chunk(c):
        def body(sb, carry):
            sl = pl.ds(c * C + sb * L, L)
            x0 = xb0[sl]
            x1 = xb1[sl]
            x2 = xb2[sl]
            d0 = db0[sl]
            d1 = db1[sl]
            inside = ((jnp.abs(x0) < SCALE / 2) & (jnp.abs(x1) < SCALE / 2)
                      & (jnp.abs(x2) < SCALE / 2))
            maskf = jnp.where(inside, jnp.float32(1.0), jnp.float32(0.0))
            iv0 = jnp.clip((x0 * (NP / SCALE) + NP / 2).astype(jnp.int32), 0, NP - 1)
            iv1 = jnp.clip((x1 * (NP / SCALE) + NP / 2).astype(jnp.int32), 0, NP - 1)
            iv2 = jnp.clip((x2 * (NP / SCALE) + NP / 2).astype(jnp.int32), 0, NP - 1)
            v = (iv0 * NP + iv1) * NP + iv2
            w = v * ROW
            gr = lax.shift_right_logical(w, 4)
            obuf[sl] = w & 15
            rows3 = (c * C + sb * L + lanes) * 3
            for k in range(3):
                gk = jnp.minimum(gr + k, jnp.int32(NGRAN - 1))
                plsc.store_scatter(ibuf, [rows3 + k], gk)
            u0 = jnp.clip((d0 * ND).astype(jnp.int32), 0, ND - 1)
            u1 = jnp.clip((d1 * ND).astype(jnp.int32), 0, ND - 1)
            ubuf[sl] = u0 * ND + u1
            mbuf[sl] = maskf
            return carry

        lax.fori_loop(0, C // L, body, 0)

    def fire(c, slot):
        cp1 = pltpu.async_copy(
            gview.at[ibuf.at[pl.ds(c * C * 3, C * 3)]], subuf.at[slot],
            semg[slot])
        cp2 = pltpu.async_copy(
            bsm.at[ubuf.at[pl.ds(c * C, C)]], bbuf.at[slot], semb[slot])
        return (cp1, cp2)

    def stage3_chunk(c, slot):
        def body(sb, carry):
            sl = pl.ds(c * C + sb * L, L)
            rows = sb * L + lanes
            m = mbuf[sl]
            fp = rows * (3 * L) + obuf[sl]

            def su_col(j):
                pos = fp + j
                return plsc.load_gather(
                    subuf, [jnp.full((L,), slot, jnp.int32),
                            lax.shift_right_logical(pos, 4), pos & 15])

            s0 = su_col(0)
            sb_out[sl] = jnp.where(m > 0.5, s0, jnp.float32(-1e30))
            bj = [plsc.load_gather(
                      bbuf, [jnp.full((L,), slot, jnp.int32), rows,
                             jnp.full((L,), j, jnp.int32)])
                  for j in range(D)]
            for cc, cb in ((0, cb0), (1, cb1), (2, cb2)):
                acc = jnp.zeros((L,), jnp.float32)
                for j in range(D):
                    sg = 1.0 / (1.0 + jnp.exp(-su_col(1 + cc * D + j)))
                    acc = acc + sg * bj[j]
                cb[sl] = acc * m
            return carry

        lax.fori_loop(0, C // L, body, 0)

    def block_body(b, carry):
        base = base0 + b * BS
        pltpu.sync_copy(x0f.at[pl.ds(base, BS)], xb0)
        pltpu.sync_copy(x1f.at[pl.ds(base, BS)], xb1)
        pltpu.sync_copy(x2f.at[pl.ds(base, BS)], xb2)
        pltpu.sync_copy(d0f.at[pl.ds(base, BS)], db0)
        pltpu.sync_copy(d1f.at[pl.ds(base, BS)], db1)
        for c in range(CPB):
            stage1_chunk(c)
        handles = {}
        for c in range(NSLOT):
            handles[c] = fire(c, c % NSLOT)
        for c in range(CPB):
            cp1, cp2 = handles.pop(c)
            cp1.wait()
            cp2.wait()
            stage3_chunk(c, c % NSLOT)
            if c + NSLOT < CPB:
                handles[c + NSLOT] = fire(c + NSLOT, c % NSLOT)
        pltpu.sync_copy(cb0, c0f.at[pl.ds(base, BS)])
        pltpu.sync_copy(cb1, c1f.at[pl.ds(base, BS)])
        pltpu.sync_copy(cb2, c2f.at[pl.ds(base, BS)])
        pltpu.sync_copy(sb_out, s0f.at[pl.ds(base, BS)])
        return carry

    lax.fori_loop(0, n_blocks, block_body, 0)


def _sc_call(x0, x1, x2, d0, d1, gview, bsm):
    n = x0.shape[0]
    mesh = plsc.VectorSubcoreMesh(core_axis_name="c", subcore_axis_name="s")
    f32 = jnp.float32
    return pl.kernel(
        _sc_body,
        out_type=[jax.ShapeDtypeStruct((n,), f32) for _ in range(4)],
        mesh=mesh,
        compiler_params=pltpu.CompilerParams(
            needs_layout_passes=False, use_tc_tiling_on_sc=False),
        scratch_types=(
            [
                pltpu.VMEM((BS,), f32),
                pltpu.VMEM((BS,), f32),
                pltpu.VMEM((BS,), f32),
                pltpu.VMEM((BS,), f32),
                pltpu.VMEM((BS,), f32),
                pltpu.VMEM((BS * 3,), jnp.int32),
                pltpu.VMEM((BS,), jnp.int32),
                pltpu.VMEM((BS,), f32),
                pltpu.VMEM((BS,), jnp.int32),
                pltpu.VMEM((NSLOT, C * 3, 16), f32),
                pltpu.VMEM((NSLOT, C, D), f32),
                pltpu.VMEM((BS,), f32),
                pltpu.VMEM((BS,), f32),
                pltpu.VMEM((BS,), f32),
                pltpu.VMEM((BS,), f32),
            ]
            + [pltpu.SemaphoreType.DMA] * (2 * NSLOT)
        ),
    )(x0, x1, x2, d0, d1, gview, bsm)


def _softmax_tc(beta2):
    rows = beta2.shape[0]
    blk = 2048

    def body(b_ref, o_ref):
        b = b_ref[...]
        mx = jnp.max(b, axis=-1, keepdims=True)
        e = jnp.exp(b - mx)
        o_ref[...] = e / jnp.sum(e, axis=-1, keepdims=True)

    return pl.pallas_call(
        body,
        grid=(rows // blk,),
        in_specs=[pl.BlockSpec((blk, D), lambda i: (i, 0))],
        out_specs=pl.BlockSpec((blk, D), lambda i: (i, 0)),
        out_shape=jax.ShapeDtypeStruct((rows, D), jnp.float32),
    )(beta2)


def _softplus_tc(s0):
    n = s0.shape[0]
    cols = 1024
    rows = n // cols
    blk = rows // 8

    def body(z_ref, o_ref):
        z = z_ref[...]
        o_ref[...] = jnp.maximum(z, 0.0) + jnp.log1p(jnp.exp(-jnp.abs(z)))

    out = pl.pallas_call(
        body,
        grid=(8,),
        in_specs=[pl.BlockSpec((blk, cols), lambda i: (i, 0))],
        out_specs=pl.BlockSpec((blk, cols), lambda i: (i, 0)),
        out_shape=jax.ShapeDtypeStruct((rows, cols), jnp.float32),
    )(s0.reshape(rows, cols))
    return out.reshape(n, 1)


def kernel(x, d, sigma_uvw, beta):
    gview = _repack_call(jnp.transpose(sigma_uvw, (0, 3, 1, 2)))
    beta2 = beta.reshape(ND * ND, D)
    bsm = _softmax_tc(beta2)
    c0, c1, c2, s0 = _sc_call(
        x[:, 0], x[:, 1], x[:, 2], d[:, 0], d[:, 1], gview, bsm)
    color = jnp.stack([c0, c1, c2], axis=-1)
    sigma = _softplus_tc(s0)
    return (color, sigma)


# speedup vs baseline: 22.7381x; 1.1420x over previous
import functools

import jax
import jax.numpy as jnp
from jax import lax
from jax.experimental import pallas as pl
from jax.experimental.pallas import tpu as pltpu
from jax.experimental.pallas import tpu_sc as plsc

SCALE = 4.0
NP = 128
ND = 128
D = 8
ROW = 1 + 3 * D

NC = 2
NS = 16
NW = NC * NS
L = 16

C = 128
NGRAN = NP * NP * NP * ROW // 16


BS = 1024
CPB = BS // C
NSLOT = 4


RB = 8
RS = RB * NP
RBLK = NP * NP // RS * 4
RROWS = RS * ROW // 16


def _repack_body(st, tact, tbuf, obuf, semi0, semi1, semo0, semo1):
    wid = lax.axis_index("s") * NC + lax.axis_index("c")
    lanes = lax.iota(jnp.int32, L)
    l25 = lanes * ROW
    semi = (semi0, semi1)
    semo = (semo0, semo1)

    def src_slice(blk):
        a = wid * 4 + lax.div(blk, RBLK // 4)
        b0 = lax.rem(blk, RBLK // 4) * RB
        return st.at[a, :, pl.ds(b0, RB), :]

    def dst_slice(blk):
        r0 = (wid * RBLK + blk) * RROWS
        return tact.at[pl.ds(r0, RROWS)]

    def fire_in(blk, s):
        pltpu.async_copy(src_slice(blk), tbuf.at[s], semi[s])

    def wait_in(s):
        pltpu.make_async_copy(src_slice(0), tbuf.at[s], semi[s]).wait()

    def fire_out(blk, s):
        pltpu.async_copy(obuf.at[s], dst_slice(blk), semo[s])

    def drain_out(s):
        pltpu.make_async_copy(obuf.at[s], dst_slice(0), semo[s]).wait()

    def compute(s):
        ob = obuf.at[s]

        def grp_body(grp, carry):
            p0 = grp * L
            pb = lax.shift_right_logical(p0, 7)
            pc = p0 & 127
            t0 = l25 + p0 * ROW
            for e in range(ROW):
                val = tbuf[s, e, pb, pl.ds(pc, L)]
                t = t0 + e
                plsc.store_scatter(
                    ob, [lax.shift_right_logical(t, 4), t & 15], val)
            return carry

        lax.fori_loop(0, RS // L, grp_body, 0)

    fire_in(0, 0)

    def pair_body(j, carry):
        blk0 = j * 2
        fire_in(blk0 + 1, 1)
        wait_in(0)

        @pl.when(j > 0)
        def _():
            drain_out(0)

        compute(0)
        fire_out(blk0, 0)

        @pl.when(j < RBLK // 2 - 1)
        def _():
            fire_in(blk0 + 2, 0)

        wait_in(1)

        @pl.when(j > 0)
        def _():
            drain_out(1)

        compute(1)
        fire_out(blk0 + 1, 1)
        return carry

    lax.fori_loop(0, RBLK // 2, pair_body, 0)
    drain_out(0)
    drain_out(1)


def _repack_call(st):
    mesh = plsc.VectorSubcoreMesh(core_axis_name="c", subcore_axis_name="s")
    f32 = jnp.float32
    return pl.kernel(
        _repack_body,
        out_type=[jax.ShapeDtypeStruct((NGRAN, 16), f32)],
        mesh=mesh,
        compiler_params=pltpu.CompilerParams(
            needs_layout_passes=False, use_tc_tiling_on_sc=False),
        scratch_types=[
            pltpu.VMEM((2, ROW, RB, NP), f32),
            pltpu.VMEM((2, RROWS, 16), f32),
            pltpu.SemaphoreType.DMA,
            pltpu.SemaphoreType.DMA,
            pltpu.SemaphoreType.DMA,
            pltpu.SemaphoreType.DMA,
        ],
    )(st)[0]


def _sc_body(x0f, x1f, x2f, d0f, d1f, gview, bsm,
             c0f, c1f, c2f, s0f,
             xb0, xb1, xb2, db0, db1, ibuf, ubuf, mbuf, obuf,
             subuf, bbuf, cb0, cb1, cb2, sb_out,
             *sems):
    semg = sems[:NSLOT]
    semb = sems[NSLOT:]
    n = s0f.shape[0]
    pts_per_tile = n // NW
    n_blocks = pts_per_tile // BS
    wid = lax.axis_index("s") * NC + lax.axis_index("c")
    base0 = wid * pts_per_tile
    lanes = lax.iota(jnp.int32, L)

    def stage1_chunk(c):
        def body(sb, carry):
            sl = pl.ds(c * C + sb * L, L)
            x0 = xb0[sl]
            x1 = xb1[sl]
            x2 = xb2[sl]
            d0 = db0[sl]
            d1 = db1[sl]
            inside = ((jnp.abs(x0) < SCALE / 2) & (jnp.abs(x1) < SCALE / 2)
                      & (jnp.abs(x2) < SCALE / 2))
            maskf = jnp.where(inside, jnp.float32(1.0), jnp.float32(0.0))
            iv0 = jnp.clip((x0 * (NP / SCALE) + NP / 2).astype(jnp.int32), 0, NP - 1)
            iv1 = jnp.clip((x1 * (NP / SCALE) + NP / 2).astype(jnp.int32), 0, NP - 1)
            iv2 = jnp.clip((x2 * (NP / SCALE) + NP / 2).astype(jnp.int32), 0, NP - 1)
            v = (iv0 * NP + iv1) * NP + iv2
            w = v * ROW
            gr = lax.shift_right_logical(w, 4)
            obuf[sl] = w & 15
            rows3 = (c * C + sb * L + lanes) * 3
            for k in range(3):
                gk = jnp.minimum(gr + k, jnp.int32(NGRAN - 1))
                plsc.store_scatter(ibuf, [rows3 + k], gk)
            u0 = jnp.clip((d0 * ND).astype(jnp.int32), 0, ND - 1)
            u1 = jnp.clip((d1 * ND).astype(jnp.int32), 0, ND - 1)
            ubuf[sl] = u0 * ND + u1
            mbuf[sl] = maskf
            return carry

        lax.fori_loop(0, C // L, body, 0)

    def fire(c, slot):
        cp1 = pltpu.async_copy(
            gview.at[ibuf.at[pl.ds(c * C * 3, C * 3)]], subuf.at[slot],
            semg[slot])
        cp2 = pltpu.async_copy(
            bsm.at[ubuf.at[pl.ds(c * C, C)]], bbuf.at[slot], semb[slot])
        return (cp1, cp2)

    def stage3_chunk(c, slot):
        sub = subuf.at[slot]
        bb = bbuf.at[slot]

        def body(sb, carry):
            sl = pl.ds(c * C + sb * L, L)
            rows = sb * L + lanes
            m = mbuf[sl]
            fp = rows * (3 * L) + obuf[sl]

            def su_col(j):
                pos = fp + j
                return plsc.load_gather(
                    sub, [lax.shift_right_logical(pos, 4), pos & 15])

            s0 = su_col(0)
            sb_out[sl] = jnp.where(m > 0.5, s0, jnp.float32(-1e30))
            bj = [plsc.load_gather(bb, [rows, jnp.full((L,), j, jnp.int32)])
                  for j in range(D)]
            for cc, cb in ((0, cb0), (1, cb1), (2, cb2)):
                acc = jnp.zeros((L,), jnp.float32)
                for j in range(D):
                    sg = 1.0 / (1.0 + jnp.exp(-su_col(1 + cc * D + j)))
                    acc = acc + sg * bj[j]
                cb[sl] = acc * m
            return carry

        lax.fori_loop(0, C // L, body, 0)

    def block_body(b, carry):
        base = base0 + b * BS
        pltpu.sync_copy(x0f.at[pl.ds(base, BS)], xb0)
        pltpu.sync_copy(x1f.at[pl.ds(base, BS)], xb1)
        pltpu.sync_copy(x2f.at[pl.ds(base, BS)], xb2)
        pltpu.sync_copy(d0f.at[pl.ds(base, BS)], db0)
        pltpu.sync_copy(d1f.at[pl.ds(base, BS)], db1)
        for c in range(CPB):
            stage1_chunk(c)
        handles = {}
        for c in range(NSLOT):
            handles[c] = fire(c, c % NSLOT)
        for c in range(CPB):
            cp1, cp2 = handles.pop(c)
            cp1.wait()
            cp2.wait()
            stage3_chunk(c, c % NSLOT)
            if c + NSLOT < CPB:
                handles[c + NSLOT] = fire(c + NSLOT, c % NSLOT)
        pltpu.sync_copy(cb0, c0f.at[pl.ds(base, BS)])
        pltpu.sync_copy(cb1, c1f.at[pl.ds(base, BS)])
        pltpu.sync_copy(cb2, c2f.at[pl.ds(base, BS)])
        pltpu.sync_copy(sb_out, s0f.at[pl.ds(base, BS)])
        return carry

    lax.fori_loop(0, n_blocks, block_body, 0)


def _sc_call(x0, x1, x2, d0, d1, gview, bsm):
    n = x0.shape[0]
    mesh = plsc.VectorSubcoreMesh(core_axis_name="c", subcore_axis_name="s")
    f32 = jnp.float32
    return pl.kernel(
        _sc_body,
        out_type=[jax.ShapeDtypeStruct((n,), f32) for _ in range(4)],
        mesh=mesh,
        compiler_params=pltpu.CompilerParams(
            needs_layout_passes=False, use_tc_tiling_on_sc=False),
        scratch_types=(
            [
                pltpu.VMEM((BS,), f32),
                pltpu.VMEM((BS,), f32),
                pltpu.VMEM((BS,), f32),
                pltpu.VMEM((BS,), f32),
                pltpu.VMEM((BS,), f32),
                pltpu.VMEM((BS * 3,), jnp.int32),
                pltpu.VMEM((BS,), jnp.int32),
                pltpu.VMEM((BS,), f32),
                pltpu.VMEM((BS,), jnp.int32),
                pltpu.VMEM((NSLOT, C * 3, 16), f32),
                pltpu.VMEM((NSLOT, C, D), f32),
                pltpu.VMEM((BS,), f32),
                pltpu.VMEM((BS,), f32),
                pltpu.VMEM((BS,), f32),
                pltpu.VMEM((BS,), f32),
            ]
            + [pltpu.SemaphoreType.DMA] * (2 * NSLOT)
        ),
    )(x0, x1, x2, d0, d1, gview, bsm)


def _softmax_tc(beta2):
    rows = beta2.shape[0]
    blk = 2048

    def body(b_ref, o_ref):
        b = b_ref[...]
        mx = jnp.max(b, axis=-1, keepdims=True)
        e = jnp.exp(b - mx)
        o_ref[...] = e / jnp.sum(e, axis=-1, keepdims=True)

    return pl.pallas_call(
        body,
        grid=(rows // blk,),
        in_specs=[pl.BlockSpec((blk, D), lambda i: (i, 0))],
        out_specs=pl.BlockSpec((blk, D), lambda i: (i, 0)),
        out_shape=jax.ShapeDtypeStruct((rows, D), jnp.float32),
    )(beta2)


def _softplus_tc(s0):
    n = s0.shape[0]
    cols = 1024
    rows = n // cols
    blk = rows // 8

    def body(z_ref, o_ref):
        z = z_ref[...]
        o_ref[...] = jnp.maximum(z, 0.0) + jnp.log1p(jnp.exp(-jnp.abs(z)))

    out = pl.pallas_call(
        body,
        grid=(8,),
        in_specs=[pl.BlockSpec((blk, cols), lambda i: (i, 0))],
        out_specs=pl.BlockSpec((blk, cols), lambda i: (i, 0)),
        out_shape=jax.ShapeDtypeStruct((rows, cols), jnp.float32),
    )(s0.reshape(rows, cols))
    return out.reshape(n, 1)


def kernel(x, d, sigma_uvw, beta):
    gview = _repack_call(jnp.transpose(sigma_uvw, (0, 3, 1, 2)))
    beta2 = beta.reshape(ND * ND, D)
    bsm = _softmax_tc(beta2)
    c0, c1, c2, s0 = _sc_call(
        x[:, 0], x[:, 1], x[:, 2], d[:, 0], d[:, 1], gview, bsm)
    color = jnp.stack([c0, c1, c2], axis=-1)
    sigma = _softplus_tc(s0)
    return (color, sigma)
